# baseline (speedup 1.0000x reference)
import functools
import math

import jax
import jax.numpy as jnp
from jax import lax
from jax.experimental import pallas as pl
from jax.experimental.pallas import tpu as pltpu
from jax.experimental.pallas import tpu_sc as plsc

_NSPEC = 89
_DIM = 128
_SCALE = 1.0 / math.sqrt(89.0)

_NC = 2
_NS = 16
_NW = _NC * _NS

_N = 100000
_C = 128
_NFULL = _N // _C
_TAIL = _N - _NFULL * _C
_TAIL_OFF = _NFULL * _C
_MAXCHUNKS = _NFULL // _NW + 1
_EXTRA = _NFULL % _NW
_IDXPAD = (_NFULL + 2) * _C
_NB = 6


def _scale_body(t_ref, o_ref):
    o_ref[...] = t_ref[...] * _SCALE


_scale_call = pl.pallas_call(
    _scale_body,
    out_shape=jax.ShapeDtypeStruct((_NSPEC, _DIM), jnp.float32),
)

_mesh = plsc.VectorSubcoreMesh(core_axis_name="c", subcore_axis_name="s")


@functools.partial(
    pl.kernel,
    out_type=jax.ShapeDtypeStruct((_N, _DIM), jnp.float32),
    mesh=_mesh,
    scratch_types=[
        pltpu.VMEM((_MAXCHUNKS * _C,), jnp.int32),
        pltpu.VMEM((_C,), jnp.int32),
        pltpu.VMEM_SHARED((_NSPEC, _DIM), jnp.float32),
        pltpu.VMEM((_NB, _C, _DIM), jnp.float32),
    ]
    + [pltpu.SemaphoreType.DMA] * (2 * _NB + 1),
)
def _gather(idx_hbm, table_hbm, out_hbm, idx_all, idx_t, table_v, rows, *sems):
    gsem = sems[:_NB]
    wsem = sems[_NB:2 * _NB]
    tsem = sems[2 * _NB]

    wid = lax.axis_index("s") * _NC + lax.axis_index("c")
    start = wid * (_MAXCHUNKS - 1) + jnp.minimum(wid, _EXTRA)
    count = (_MAXCHUNKS - 1) + (wid < _EXTRA).astype(jnp.int32)

    @pl.when(lax.axis_index("s") == 0)
    def _():
        pltpu.sync_copy(table_hbm, table_v)

    plsc.subcore_barrier()
    pltpu.sync_copy(idx_hbm.at[pl.ds(start * _C, _MAXCHUNKS * _C)], idx_all)

    def _wait_gather(b):
        pltpu.make_async_copy(
            out_hbm.at[pl.ds(0, _C)], rows.at[b], gsem[b]
        ).wait()

    def _wait_write(b):
        pltpu.make_async_copy(
            rows.at[b], out_hbm.at[pl.ds(0, _C)], wsem[b]
        ).wait()

    for b in range(_NB):
        pltpu.async_copy(
            table_v.at[idx_all.at[pl.ds(b * _C, _C)]], rows.at[b], gsem[b]
        )

    def body(j, carry):
        for b in range(_NB):
            k = j * _NB + b

            @pl.when(k < count)
            def _():
                _wait_gather(b)
                off = (start + k) * _C
                pltpu.async_copy(
                    rows.at[b], out_hbm.at[pl.ds(off, _C)], wsem[b]
                )
                bp = (b - 1) % _NB

                @pl.when(k >= 1)
                def _():
                    _wait_write(bp)
                    kp = k - 1 + _NB

                    @pl.when(kp < count)
                    def _():
                        pltpu.async_copy(
                            table_v.at[idx_all.at[pl.ds(kp * _C, _C)]],
                            rows.at[bp],
                            gsem[bp],
                        )

        return carry

    lax.fori_loop(0, (_MAXCHUNKS + _NB - 1) // _NB, body, 0)

    last = count - 1
    for b in range(_NB):
        @pl.when(last % _NB == b)
        def _():
            _wait_write(b)

    @pl.when(wid == _NW - 1)
    def _():
        pltpu.sync_copy(idx_hbm.at[pl.ds(_TAIL_OFF, _C)], idx_t)
        pltpu.async_copy(table_v.at[idx_t], rows.at[0], tsem).wait()
        pltpu.sync_copy(
            rows.at[0].at[pl.ds(0, _TAIL)], out_hbm.at[pl.ds(_TAIL_OFF, _TAIL)]
        )


def kernel(node_specie, embeddings):
    idx = node_specie.astype(jnp.int32)
    idx = jnp.pad(idx, (0, _IDXPAD - _N))
    scaled = _scale_call(embeddings)
    return _gather(idx, scaled)


# speedup vs baseline: 1.0371x; 1.0371x over previous
import functools
import math

import jax
import jax.numpy as jnp
from jax import lax
from jax.experimental import pallas as pl
from jax.experimental.pallas import tpu as pltpu
from jax.experimental.pallas import tpu_sc as plsc

_NSPEC = 89
_DIM = 128
_SCALE = 1.0 / math.sqrt(89.0)

_NC = 2
_NS = 16
_NW = _NC * _NS

_N = 100000
_C = 128
_NFULL = _N // _C
_TAIL = _N - _NFULL * _C
_TAIL_OFF = _NFULL * _C
_MAXCHUNKS = _NFULL // _NW + 1
_EXTRA = _NFULL % _NW
_PRELOAD = _MAXCHUNKS * _C
_SHIFT = 96
_NB = 4
_NBLK = (_NSPEC + 7) // 8

_mesh = plsc.VectorSubcoreMesh(core_axis_name="c", subcore_axis_name="s")


@functools.partial(
    pl.kernel,
    out_type=jax.ShapeDtypeStruct((_N, _DIM), jnp.float32),
    mesh=_mesh,
    scratch_types=[
        pltpu.VMEM((_PRELOAD,), jnp.int32),
        pltpu.VMEM((8, _DIM), jnp.float32),
        pltpu.VMEM_SHARED((_NSPEC, _DIM), jnp.float32),
        pltpu.VMEM((_NB, _C, _DIM), jnp.float32),
    ]
    + [pltpu.SemaphoreType.DMA] * (2 * _NB + 1),
)
def _gather(idx_hbm, table_hbm, out_hbm, idx_all, blk_v, table_v, rows, *sems):
    gsem = sems[:_NB]
    wsem = sems[_NB:2 * _NB]
    tsem = sems[2 * _NB]

    sid = lax.axis_index("s")
    wid = sid * _NC + lax.axis_index("c")
    start = wid * (_MAXCHUNKS - 1) + jnp.minimum(wid, _EXTRA)
    is_last = wid == _NW - 1
    shift = jnp.where(is_last, _SHIFT, 0)
    count = (_MAXCHUNKS - 1) + (wid < _EXTRA).astype(jnp.int32)

    @pl.when(sid < _NBLK - 1)
    def _():
        r0 = sid * 8
        pltpu.sync_copy(table_hbm.at[pl.ds(r0, 8)], blk_v)
        for r in range(8):
            for j in range(_DIM // 16):
                blk_v[r, pl.ds(j * 16, 16)] = (
                    blk_v[r, pl.ds(j * 16, 16)] * _SCALE
                )
        pltpu.sync_copy(blk_v, table_v.at[pl.ds(r0, 8)])

    @pl.when(sid == _NBLK - 1)
    def _():
        r0 = (_NBLK - 1) * 8
        pltpu.sync_copy(table_hbm.at[pl.ds(r0, 1)], blk_v.at[pl.ds(0, 1)])
        for j in range(_DIM // 16):
            blk_v[0, pl.ds(j * 16, 16)] = blk_v[0, pl.ds(j * 16, 16)] * _SCALE
        pltpu.sync_copy(blk_v.at[pl.ds(0, 1)], table_v.at[pl.ds(r0, 1)])

    pltpu.sync_copy(idx_hbm.at[pl.ds(start * _C - shift, _PRELOAD)], idx_all)
    plsc.subcore_barrier()

    def _wait_gather(b):
        pltpu.make_async_copy(
            out_hbm.at[pl.ds(0, _C)], rows.at[b], gsem[b]
        ).wait()

    def _wait_write(b):
        pltpu.make_async_copy(
            rows.at[b], out_hbm.at[pl.ds(0, _C)], wsem[b]
        ).wait()

    def _issue_gather(k, b):
        pltpu.async_copy(
            table_v.at[idx_all.at[pl.ds(shift + k * _C, _C)]],
            rows.at[b],
            gsem[b],
        )

    for b in range(_NB):
        _issue_gather(b, b)

    def body(j, carry):
        for b in range(_NB):
            k = j * _NB + b

            @pl.when(k < count)
            def _():
                _wait_gather(b)
                off = (start + k) * _C
                pltpu.async_copy(
                    rows.at[b], out_hbm.at[pl.ds(off, _C)], wsem[b]
                )
                bp = (b - 1) % _NB

                @pl.when(k >= 1)
                def _():
                    _wait_write(bp)
                    kp = k - 1 + _NB

                    @pl.when(kp < count)
                    def _():
                        _issue_gather(kp, bp)

        return carry

    lax.fori_loop(0, (_MAXCHUNKS + _NB - 1) // _NB, body, 0)

    last = count - 1
    for b in range(_NB):
        @pl.when(last % _NB == b)
        def _():
            _wait_write(b)

    @pl.when(is_last)
    def _():
        pltpu.async_copy(
            table_v.at[idx_all.at[pl.ds(_PRELOAD - _TAIL, _TAIL)]],
            rows.at[0].at[pl.ds(0, _TAIL)],
            tsem,
        ).wait()
        pltpu.sync_copy(
            rows.at[0].at[pl.ds(0, _TAIL)], out_hbm.at[pl.ds(_TAIL_OFF, _TAIL)]
        )


def kernel(node_specie, embeddings):
    return _gather(node_specie.astype(jnp.int32), embeddings)
